# baseline (speedup 1.0000x reference)
import functools

import jax
import jax.numpy as jnp
from jax import lax
from jax.experimental import pallas as pl
from jax.experimental.pallas import tpu as pltpu
from jax.experimental.pallas import tpu_sc as plsc

_LANE = 128
_K = 8


@functools.cache
def _make_lookup(B: int, D: int):
    info = plsc.get_sparse_core_info()
    NC, NS = info.num_cores, info.num_subcores
    NW = NC * NS
    chunk = _K * _LANE
    assert B % (NW * chunk) == 0, (B, NW, chunk)
    rows_per_w = B // NW // _LANE
    G = rows_per_w // _K
    mesh = plsc.VectorSubcoreMesh(core_axis_name="c", subcore_axis_name="s")

    @functools.partial(
        pl.kernel,
        out_type=jax.ShapeDtypeStruct((B, D), jnp.float32),
        mesh=mesh,
        scratch_types=[
            pltpu.VMEM((_K, _LANE), jnp.int32),
            pltpu.VMEM((chunk, D), jnp.float32),
            pltpu.SemaphoreType.DMA,
        ],
        compiler_params=pltpu.CompilerParams(use_tc_tiling_on_sc=False),
    )
    def lookup(table_hbm, idx_hbm, out_hbm, idx_v, rows_v, sem):
        wid = lax.axis_index("s") * NC + lax.axis_index("c")
        row0 = wid * rows_per_w

        @pl.loop(0, G)
        def _chunk(g):
            r = row0 + g * _K
            pltpu.sync_copy(idx_hbm.at[pl.ds(r, _K)], idx_v)
            copies = []
            for j in range(_K):
                copies.append(
                    pltpu.async_copy(
                        table_hbm.at[idx_v.at[j]],
                        rows_v.at[pl.ds(j * _LANE, _LANE)],
                        sem,
                    )
                )
            for c in copies:
                c.wait()
            pltpu.sync_copy(rows_v, out_hbm.at[pl.ds(r * _LANE, chunk)])

    return lookup


def kernel(x, table):
    B = x.size
    D = table.shape[1]
    idx2d = x.reshape(B // _LANE, _LANE)
    out = _make_lookup(B, D)(table, idx2d)
    return out.reshape(x.shape + (D,))


# speedup vs baseline: 1.0186x; 1.0186x over previous
import functools

import jax
import jax.numpy as jnp
from jax import lax
from jax.experimental import pallas as pl
from jax.experimental.pallas import tpu as pltpu
from jax.experimental.pallas import tpu_sc as plsc

_LANE = 128
_K = 5


@functools.cache
def _make_lookup(B: int, D: int):
    info = plsc.get_sparse_core_info()
    NC, NS = info.num_cores, info.num_subcores
    NW = NC * NS
    chunk = _K * _LANE
    assert B % (NW * chunk) == 0, (B, NW, chunk)
    rows_per_w = B // NW // _LANE
    G = rows_per_w // _K
    mesh = plsc.VectorSubcoreMesh(core_axis_name="c", subcore_axis_name="s")

    @functools.partial(
        pl.kernel,
        out_type=jax.ShapeDtypeStruct((B, D), jnp.float32),
        mesh=mesh,
        scratch_types=[
            pltpu.VMEM((rows_per_w, _LANE), jnp.int32),
            pltpu.VMEM((2, chunk, D), jnp.float32),
            pltpu.SemaphoreType.DMA,
            pltpu.SemaphoreType.DMA,
        ],
        compiler_params=pltpu.CompilerParams(use_tc_tiling_on_sc=False),
    )
    def lookup(table_hbm, idx_hbm, out_hbm, idx_v, rows_v, gsem, osem):
        wid = lax.axis_index("s") * NC + lax.axis_index("c")
        row0 = wid * rows_per_w
        pltpu.sync_copy(idx_hbm.at[pl.ds(row0, rows_per_w)], idx_v)

        def fire_gathers(g, b):
            for j in range(_K):
                pltpu.async_copy(
                    table_hbm.at[idx_v.at[g * _K + j]],
                    rows_v.at[b].at[pl.ds(j * _LANE, _LANE)],
                    gsem,
                )

        def drain(sem, b):
            pltpu.make_async_copy(
                out_hbm.at[pl.ds(0, chunk)], rows_v.at[b], sem
            ).wait()

        def fire_out(g, b):
            pltpu.async_copy(
                rows_v.at[b],
                out_hbm.at[pl.ds((row0 + g * _K) * _LANE, chunk)],
                osem,
            )

        fire_gathers(0, 0)

        @pl.loop(1, G)
        def _chunk(g):
            b = lax.rem(g, 2)
            @pl.when(g >= 2)
            def _():
                drain(osem, b)

            fire_gathers(g, b)
            drain(gsem, 1 - b)
            fire_out(g - 1, 1 - b)

        last = (G - 1) % 2
        drain(gsem, last)
        fire_out(G - 1, last)
        drain(osem, 1 - last)
        drain(osem, last)

    return lookup


def kernel(x, table):
    B = x.size
    D = table.shape[1]
    idx2d = x.reshape(B // _LANE, _LANE)
    out = _make_lookup(B, D)(table, idx2d)
    return out.reshape(x.shape + (D,))


# speedup vs baseline: 1.2419x; 1.2192x over previous
import functools

import jax
import jax.numpy as jnp
from jax import lax
from jax.experimental import pallas as pl
from jax.experimental.pallas import tpu as pltpu
from jax.experimental.pallas import tpu_sc as plsc

_DP = 128


@functools.cache
def _make_lookup(BATCH: int, SEQ: int, V: int):
    info = plsc.get_sparse_core_info()
    NC, NS = info.num_cores, info.num_subcores
    NW = NC * NS
    assert BATCH % NW == 0
    b_per_w = BATCH // NW
    n_idx = b_per_w * SEQ
    slabs = []
    off = 0
    while off < SEQ:
        n = min(128, SEQ - off)
        slabs.append((off, n))
        off += n
    mesh = plsc.VectorSubcoreMesh(core_axis_name="c", subcore_axis_name="s")

    @functools.partial(
        pl.kernel,
        out_type=jax.ShapeDtypeStruct((BATCH, SEQ, _DP), jnp.float32),
        mesh=mesh,
        scratch_types=[
            pltpu.VMEM((n_idx,), jnp.int32),
            pltpu.VMEM((2, SEQ, _DP), jnp.float32),
            pltpu.SemaphoreType.DMA,
            pltpu.SemaphoreType.DMA,
        ],
        compiler_params=pltpu.CompilerParams(use_tc_tiling_on_sc=True),
    )
    def lookup(table_hbm, idx_hbm, out_hbm, idx_v, rows_v, gsem, osem):
        wid = lax.axis_index("s") * NC + lax.axis_index("c")
        b0 = wid * b_per_w
        pltpu.sync_copy(idx_hbm.at[pl.ds(b0 * SEQ, n_idx)], idx_v)

        def fire_gathers(i, b):
            for off, n in slabs:
                pltpu.async_copy(
                    table_hbm.at[idx_v.at[pl.ds(i * SEQ + off, n)]],
                    rows_v.at[b].at[pl.ds(off, n)],
                    gsem,
                )

        def drain(sem):
            pltpu.make_async_copy(
                out_hbm.at[b0], rows_v.at[0], sem
            ).wait()

        def fire_out(i, b):
            pltpu.async_copy(rows_v.at[b], out_hbm.at[b0 + i], osem)

        fire_gathers(0, 0)

        @pl.loop(1, b_per_w)
        def _chunk(i):
            b = lax.rem(i, 2)

            @pl.when(i >= 2)
            def _():
                drain(osem)

            fire_gathers(i, b)
            drain(gsem)
            fire_out(i - 1, 1 - b)

        last = (b_per_w - 1) % 2
        drain(gsem)
        fire_out(b_per_w - 1, last)
        drain(osem)
        drain(osem)

    return lookup


def kernel(x, table):
    BATCH, SEQ = x.shape
    V, D = table.shape
    table_p = jnp.pad(table, ((0, 0), (0, _DP - D)))
    out_p = _make_lookup(BATCH, SEQ, V)(table_p, x.reshape(-1))
    return out_p[:, :, :D]
